# Initial kernel scaffold; baseline (speedup 1.0000x reference)
import jax
import jax.numpy as jnp
from jax.experimental import pallas as pl


def kernel(x, edge_attr, Wmlp, bmlp, We1, We2, gamma, beta, edge_index):
    raise NotImplementedError("write your pallas kernel here")


# speedup vs baseline: 6.4659x; 6.4659x over previous
import dataclasses
import functools

import jax
import jax.numpy as jnp
from jax import lax
from jax.experimental import pallas as pl
from jax.experimental.pallas import tpu as pltpu
from jax.experimental.pallas import tpu_sc as plsc

N = 10000
E = 320000
D = 128
L = 5
EPS = 1e-5

NC = 2
NS = 16
NW = NC * NS
EPW = E // NW
CH = 128
NFULL = EPW // CH
TAIL = EPW - NFULL * CH
NPAD = 10240
RPT = NPAD // NS

_mesh = plsc.VectorSubcoreMesh(core_axis_name="c", subcore_axis_name="s")

_cp_no_layout = pltpu.CompilerParams()
if "needs_layout_passes" in pltpu.CompilerParams.__dataclass_fields__:
    _cp_no_layout = dataclasses.replace(_cp_no_layout, needs_layout_passes=False)


def _zero_shared(shared, z_hbm, sid):
    pltpu.sync_copy(z_hbm, shared.at[pl.ds(sid * RPT, RPT)])


def _writeout_shared(shared, out_hbm, cid, sid):
    pltpu.sync_copy(shared.at[pl.ds(sid * RPT, RPT)],
                    out_hbm.at[cid, pl.ds(sid * RPT, RPT)])


def _sc_deg(col, zn):
    @functools.partial(
        pl.kernel,
        out_type=jax.ShapeDtypeStruct((NW, NPAD), jnp.float32),
        mesh=_mesh,
        scratch_types=[
            pltpu.VMEM((CH,), jnp.int32),
            pltpu.VMEM((NPAD,), jnp.float32),
        ],
        compiler_params=_cp_no_layout,
    )
    def k(col_hbm, z_hbm, out_hbm, idx_v, acc_v):
        cid = lax.axis_index("c")
        sid = lax.axis_index("s")
        wid = sid * NC + cid
        base = wid * EPW

        pltpu.sync_copy(z_hbm, acc_v)
        ones = jnp.full((16,), 1.0, jnp.float32)

        def do_chunk(off, n):
            pltpu.sync_copy(col_hbm.at[pl.ds(off, n)], idx_v.at[pl.ds(0, n)])
            for grp in range(n // 16):
                cv = idx_v[pl.ds(grp * 16, 16)]
                plsc.addupdate_scatter(acc_v, [cv], ones)

        @pl.loop(0, NFULL)
        def _(c):
            do_chunk(base + c * CH, CH)

        if TAIL:
            do_chunk(base + NFULL * CH, TAIL)

        pltpu.sync_copy(acc_v, out_hbm.at[wid])

    return k(col, zn)


def _sc_bacc(ea_pad, row, col, dinv1d, z4):
    @functools.partial(
        pl.kernel,
        out_type=jax.ShapeDtypeStruct((NW, 4 * NPAD), jnp.float32),
        mesh=_mesh,
        scratch_types=[
            pltpu.VMEM((CH,), jnp.int32),
            pltpu.VMEM((CH,), jnp.int32),
            pltpu.VMEM((CH, 16), jnp.float32),
            pltpu.VMEM((NPAD,), jnp.float32),
            pltpu.VMEM((4 * NPAD,), jnp.float32),
        ],
        compiler_params=_cp_no_layout,
    )
    def k(ea_hbm, row_hbm, col_hbm, dinv_hbm, z_hbm, out_hbm, ridx_v, cidx_v,
          buf_v, dinv_v, acc_v):
        cid = lax.axis_index("c")
        sid = lax.axis_index("s")
        wid = sid * NC + cid
        base = wid * EPW

        pltpu.sync_copy(dinv_hbm, dinv_v)
        pltpu.sync_copy(z_hbm, acc_v)

        def do_chunk(off, n):
            pltpu.sync_copy(row_hbm.at[pl.ds(off, n)], ridx_v.at[pl.ds(0, n)])
            pltpu.sync_copy(col_hbm.at[pl.ds(off, n)], cidx_v.at[pl.ds(0, n)])
            pltpu.sync_copy(ea_hbm.at[pl.ds(off, n)], buf_v.at[pl.ds(0, n)])
            for grp in range(n // 16):
                rv = ridx_v[pl.ds(grp * 16, 16)]
                cv = cidx_v[pl.ds(grp * 16, 16)]
                dvals = plsc.load_gather(dinv_v, [rv])
                eidx = lax.iota(jnp.int32, 16) + grp * 16
                cv4 = cv * 4
                for f in range(4):
                    fidx = jnp.full((16,), f, jnp.int32)
                    vals = plsc.load_gather(buf_v, [eidx, fidx])
                    plsc.addupdate_scatter(acc_v, [cv4 + f], vals * dvals)

        @pl.loop(0, NFULL)
        def _(c):
            do_chunk(base + c * CH, CH)

        if TAIL:
            do_chunk(base + NFULL * CH, TAIL)

        pltpu.sync_copy(acc_v, out_hbm.at[wid])

    return k(ea_pad, row, col, dinv1d, z4)


def _sc_spmm(g, row, col, z128):
    @functools.partial(
        pl.kernel,
        out_type=jax.ShapeDtypeStruct((NC, NPAD, D), jnp.float32),
        mesh=_mesh,
        scratch_types=[
            pltpu.VMEM((CH,), jnp.int32),
            pltpu.VMEM((CH,), jnp.int32),
            pltpu.VMEM((CH, D), jnp.float32),
            pltpu.VMEM_SHARED((NPAD, D), jnp.float32),
            pltpu.SemaphoreType.DMA,
        ],
    )
    def k(g_hbm, row_hbm, col_hbm, z_hbm, out_hbm, ridx_v, cidx_v, rows_v,
          shared, sem):
        cid = lax.axis_index("c")
        sid = lax.axis_index("s")
        wid = sid * NC + cid
        base = wid * EPW

        _zero_shared(shared, z_hbm, sid)
        plsc.subcore_barrier()

        @pl.loop(0, NFULL)
        def _(c):
            pltpu.sync_copy(row_hbm.at[pl.ds(base + c * CH, CH)], ridx_v)
            pltpu.sync_copy(col_hbm.at[pl.ds(base + c * CH, CH)], cidx_v)
            pltpu.async_copy(g_hbm.at[ridx_v], rows_v, sem).wait()
            pltpu.sync_copy(rows_v, shared.at[cidx_v], add=True)

        if TAIL:
            pltpu.sync_copy(row_hbm.at[pl.ds(base + NFULL * CH, TAIL)],
                            ridx_v.at[pl.ds(0, TAIL)])
            pltpu.sync_copy(col_hbm.at[pl.ds(base + NFULL * CH, TAIL)],
                            cidx_v.at[pl.ds(0, TAIL)])
            pltpu.async_copy(g_hbm.at[ridx_v.at[pl.ds(0, TAIL)]],
                             rows_v.at[pl.ds(0, TAIL)], sem).wait()
            pltpu.sync_copy(rows_v.at[pl.ds(0, TAIL)],
                            shared.at[cidx_v.at[pl.ds(0, TAIL)]], add=True)

        plsc.subcore_barrier()
        _writeout_shared(shared, out_hbm, cid, sid)

    return k(g, row, col, z128)


def _tc_prep(deg_parts, x):
    def body(degp_ref, x_ref, dinv1d_ref, dinvc_ref, g0_ref):
        deg = jnp.sum(degp_ref[...], axis=0, keepdims=True)
        dinv = jnp.where(deg > 0, lax.rsqrt(deg), 0.0)
        dinv1d_ref[...] = dinv.reshape(NPAD)
        dinvc = dinv.reshape(NPAD, 1)[:N]
        dinvc_ref[...] = dinvc
        g0_ref[...] = x_ref[...] * dinvc

    return pl.pallas_call(
        body,
        out_shape=(jax.ShapeDtypeStruct((NPAD,), jnp.float32),
                   jax.ShapeDtypeStruct((N, 1), jnp.float32),
                   jax.ShapeDtypeStruct((N, D), jnp.float32)),
    )(deg_parts, x)


NB = 1000


def _tc_mlp_stats(h, agg_parts, b_parts, dinvc, wm, bm, we1r, we2r):
    def body(h_ref, aggp_ref, bp_ref, dinv_ref, wm_ref, bm_ref, we1_ref,
             we2_ref, h2_ref, st_ref):
        dinv = dinv_ref[...]
        b4 = jnp.sum(bp_ref[...], axis=0)
        ef1 = b4[:, 3:4] * we1_ref[...]
        ef2 = (b4[:, 0:1] * we2_ref[...][0:1]
               + b4[:, 1:2] * we2_ref[...][1:2]
               + b4[:, 2:3] * we2_ref[...][2:3])
        ef = jnp.concatenate([ef1, ef2], axis=1)
        agg = dinv * (aggp_ref[0] + aggp_ref[1] + ef)
        cat = jnp.concatenate([h_ref[...], agg], axis=1)
        z = lax.dot_general(cat, wm_ref[...], (((1,), (1,)), ((), ())),
                            preferred_element_type=jnp.float32)
        h2 = jnp.maximum(z + bm_ref[...], 0.0)
        h2_ref[...] = h2

        @pl.when(pl.program_id(0) == 0)
        def _():
            st_ref[...] = jnp.zeros_like(st_ref)

        st_ref[...] += jnp.concatenate(
            [jnp.sum(h2, axis=0, keepdims=True),
             jnp.sum(h2 * h2, axis=0, keepdims=True)], axis=0)

    return pl.pallas_call(
        body,
        grid=(N // NB,),
        in_specs=[
            pl.BlockSpec((NB, D), lambda i: (i, 0)),
            pl.BlockSpec((NC, NB, D), lambda i: (0, i, 0)),
            pl.BlockSpec((NW, NB, 4), lambda i: (0, i, 0)),
            pl.BlockSpec((NB, 1), lambda i: (i, 0)),
            pl.BlockSpec((D, 2 * D), lambda i: (0, 0)),
            pl.BlockSpec((1, D), lambda i: (0, 0)),
            pl.BlockSpec((1, D // 2), lambda i: (0, 0)),
            pl.BlockSpec((3, D // 2), lambda i: (0, 0)),
        ],
        out_specs=(pl.BlockSpec((NB, D), lambda i: (i, 0)),
                   pl.BlockSpec((2, D), lambda i: (0, 0))),
        out_shape=(jax.ShapeDtypeStruct((N, D), jnp.float32),
                   jax.ShapeDtypeStruct((2, D), jnp.float32)),
    )(h, agg_parts, b_parts, dinvc, wm, bm, we1r, we2r)


def _tc_bn_apply(h2, stats, dinvc, gm, bt, last):
    def body(h2_ref, st_ref, dinv_ref, gm_ref, bt_ref, hout_ref, gout_ref):
        st = st_ref[...]
        mean = st[0:1] / N
        var = st[1:2] / N - mean * mean
        hn = (h2_ref[...] - mean) / jnp.sqrt(var + EPS) * gm_ref[...] \
            + bt_ref[...]
        if not last:
            hn = jnp.maximum(hn, 0.0)
        hout_ref[...] = hn
        gout_ref[...] = hn * dinv_ref[...]

    return pl.pallas_call(
        body,
        grid=(N // NB,),
        in_specs=[
            pl.BlockSpec((NB, D), lambda i: (i, 0)),
            pl.BlockSpec((2, D), lambda i: (0, 0)),
            pl.BlockSpec((NB, 1), lambda i: (i, 0)),
            pl.BlockSpec((1, D), lambda i: (0, 0)),
            pl.BlockSpec((1, D), lambda i: (0, 0)),
        ],
        out_specs=(pl.BlockSpec((NB, D), lambda i: (i, 0)),
                   pl.BlockSpec((NB, D), lambda i: (i, 0))),
        out_shape=(jax.ShapeDtypeStruct((N, D), jnp.float32),
                   jax.ShapeDtypeStruct((N, D), jnp.float32)),
    )(h2, stats, dinvc, gm, bt)


def kernel(x, edge_attr, Wmlp, bmlp, We1, We2, gamma, beta, edge_index):
    row = edge_index[0]
    col = edge_index[1]
    ea_bf = edge_attr.astype(jnp.bfloat16).astype(jnp.float32)
    ea_pad = jnp.concatenate(
        [ea_bf, jnp.zeros((E, 12), jnp.float32)], axis=1)
    zn = jnp.zeros((NPAD,), jnp.float32)
    z4 = jnp.zeros((4 * NPAD,), jnp.float32)
    z128 = jnp.zeros((RPT, D), jnp.float32)

    deg_parts = _sc_deg(col, zn)
    dinv1d, dinvc, g = _tc_prep(deg_parts, x)
    b_parts = _sc_bacc(ea_pad, row, col, dinv1d, z4)
    b_parts = b_parts.reshape(NW, NPAD, 4)

    h = x
    for l in range(L):
        agg_parts = _sc_spmm(g, row, col, z128)
        h2, stats = _tc_mlp_stats(
            h, agg_parts, b_parts, dinvc,
            Wmlp[l], bmlp[l].reshape(1, D),
            We1[l].reshape(1, D // 2).astype(jnp.bfloat16).astype(jnp.float32),
            We2[l].T.astype(jnp.bfloat16).astype(jnp.float32))
        h, g = _tc_bn_apply(h2, stats, dinvc,
                            gamma[l].reshape(1, D), beta[l].reshape(1, D),
                            last=(l == L - 1))
    return h


# speedup vs baseline: 9.1957x; 1.4222x over previous
import dataclasses
import functools

import jax
import jax.numpy as jnp
from jax import lax
from jax.experimental import pallas as pl
from jax.experimental.pallas import tpu as pltpu
from jax.experimental.pallas import tpu_sc as plsc

N = 10000
E = 320000
D = 128
L = 5
EPS = 1e-5

NC = 2
NS = 16
NW = NC * NS
EPW = E // NW
CH = 128
NFULL = EPW // CH
TAIL = EPW - NFULL * CH
NPAD = 10240
RPT = NPAD // NS

_mesh = plsc.VectorSubcoreMesh(core_axis_name="c", subcore_axis_name="s")

_cp_no_layout = pltpu.CompilerParams()
if "needs_layout_passes" in pltpu.CompilerParams.__dataclass_fields__:
    _cp_no_layout = dataclasses.replace(_cp_no_layout, needs_layout_passes=False)


def _zero_shared(shared, z_hbm, sid):
    pltpu.sync_copy(z_hbm, shared.at[pl.ds(sid * RPT, RPT)])


def _writeout_shared(shared, out_hbm, cid, sid):
    pltpu.sync_copy(shared.at[pl.ds(sid * RPT, RPT)],
                    out_hbm.at[cid, pl.ds(sid * RPT, RPT)])


def _sc_deg(col, zn):
    @functools.partial(
        pl.kernel,
        out_type=jax.ShapeDtypeStruct((NW, NPAD), jnp.float32),
        mesh=_mesh,
        scratch_types=[
            pltpu.VMEM((EPW,), jnp.int32),
            pltpu.VMEM((NPAD,), jnp.float32),
        ],
        compiler_params=_cp_no_layout,
    )
    def k(col_hbm, z_hbm, out_hbm, idx_v, acc_v):
        cid = lax.axis_index("c")
        sid = lax.axis_index("s")
        wid = sid * NC + cid
        base = wid * EPW

        pltpu.sync_copy(z_hbm, acc_v)
        pltpu.sync_copy(col_hbm.at[pl.ds(base, EPW)], idx_v)
        ones = jnp.full((16,), 1.0, jnp.float32)

        @pl.loop(0, EPW // 16)
        def _(g):
            cv = idx_v[pl.ds(g * 16, 16)]
            plsc.addupdate_scatter(acc_v, [cv], ones)

        pltpu.sync_copy(acc_v, out_hbm.at[wid])

    return k(col, zn)


EAC = 2000


def _sc_bacc(ea_pad, row, col, dinv1d, z4):
    @functools.partial(
        pl.kernel,
        out_type=jax.ShapeDtypeStruct((NW, 4 * NPAD), jnp.float32),
        mesh=_mesh,
        scratch_types=[
            pltpu.VMEM((EPW,), jnp.int32),
            pltpu.VMEM((EPW,), jnp.int32),
            pltpu.VMEM((EAC, 16), jnp.float32),
            pltpu.VMEM((NPAD,), jnp.float32),
            pltpu.VMEM((4 * NPAD,), jnp.float32),
        ],
        compiler_params=_cp_no_layout,
    )
    def k(ea_hbm, row_hbm, col_hbm, dinv_hbm, z_hbm, out_hbm, ridx_v, cidx_v,
          buf_v, dinv_v, acc_v):
        cid = lax.axis_index("c")
        sid = lax.axis_index("s")
        wid = sid * NC + cid
        base = wid * EPW

        pltpu.sync_copy(dinv_hbm, dinv_v)
        pltpu.sync_copy(z_hbm, acc_v)
        pltpu.sync_copy(row_hbm.at[pl.ds(base, EPW)], ridx_v)
        pltpu.sync_copy(col_hbm.at[pl.ds(base, EPW)], cidx_v)

        @pl.loop(0, EPW // EAC)
        def _(ch):
            off = ch * EAC
            pltpu.sync_copy(ea_hbm.at[pl.ds(base + off, EAC)], buf_v)

            @pl.loop(0, EAC // 16)
            def _(g):
                rv = ridx_v[pl.ds(off + g * 16, 16)]
                cv = cidx_v[pl.ds(off + g * 16, 16)]
                dvals = plsc.load_gather(dinv_v, [rv])
                eidx = lax.iota(jnp.int32, 16) + g * 16
                cv4 = cv * 4
                for f in range(4):
                    fidx = jnp.full((16,), f, jnp.int32)
                    vals = plsc.load_gather(buf_v, [eidx, fidx])
                    plsc.addupdate_scatter(acc_v, [cv4 + f], vals * dvals)

        pltpu.sync_copy(acc_v, out_hbm.at[wid])

    return k(ea_pad, row, col, dinv1d, z4)


def _sc_bacc(ea_pad, row, col, dinv1d, z4):
    @functools.partial(
        pl.kernel,
        out_type=jax.ShapeDtypeStruct((NW, 4 * NPAD), jnp.float32),
        mesh=_mesh,
        scratch_types=[
            pltpu.VMEM((CH,), jnp.int32),
            pltpu.VMEM((CH,), jnp.int32),
            pltpu.VMEM((CH, 16), jnp.float32),
            pltpu.VMEM((NPAD,), jnp.float32),
            pltpu.VMEM((4 * NPAD,), jnp.float32),
        ],
        compiler_params=_cp_no_layout,
    )
    def k(ea_hbm, row_hbm, col_hbm, dinv_hbm, z_hbm, out_hbm, ridx_v, cidx_v,
          buf_v, dinv_v, acc_v):
        cid = lax.axis_index("c")
        sid = lax.axis_index("s")
        wid = sid * NC + cid
        base = wid * EPW

        pltpu.sync_copy(dinv_hbm, dinv_v)
        pltpu.sync_copy(z_hbm, acc_v)

        def do_chunk(off, n):
            pltpu.sync_copy(row_hbm.at[pl.ds(off, n)], ridx_v.at[pl.ds(0, n)])
            pltpu.sync_copy(col_hbm.at[pl.ds(off, n)], cidx_v.at[pl.ds(0, n)])
            pltpu.sync_copy(ea_hbm.at[pl.ds(off, n)], buf_v.at[pl.ds(0, n)])
            for grp in range(n // 16):
                rv = ridx_v[pl.ds(grp * 16, 16)]
                cv = cidx_v[pl.ds(grp * 16, 16)]
                dvals = plsc.load_gather(dinv_v, [rv])
                eidx = lax.iota(jnp.int32, 16) + grp * 16
                cv4 = cv * 4
                for f in range(4):
                    fidx = jnp.full((16,), f, jnp.int32)
                    vals = plsc.load_gather(buf_v, [eidx, fidx])
                    plsc.addupdate_scatter(acc_v, [cv4 + f], vals * dvals)

        @pl.loop(0, NFULL)
        def _(c):
            do_chunk(base + c * CH, CH)

        if TAIL:
            do_chunk(base + NFULL * CH, TAIL)

        pltpu.sync_copy(acc_v, out_hbm.at[wid])

    return k(ea_pad, row, col, dinv1d, z4)


NBUF = 2


def _sc_spmm(g, row, col, z128):
    @functools.partial(
        pl.kernel,
        out_type=jax.ShapeDtypeStruct((NC, NPAD, D), jnp.float32),
        mesh=_mesh,
        scratch_types=[
            pltpu.VMEM((NBUF, CH), jnp.int32),
            pltpu.VMEM((NBUF, CH), jnp.int32),
            pltpu.VMEM((NBUF, CH, D), jnp.float32),
            pltpu.VMEM((TAIL,), jnp.int32),
            pltpu.VMEM((TAIL,), jnp.int32),
            pltpu.VMEM_SHARED((NPAD, D), jnp.float32),
            pltpu.SemaphoreType.DMA,
            pltpu.SemaphoreType.DMA,
            pltpu.SemaphoreType.DMA,
            pltpu.SemaphoreType.DMA,
            pltpu.SemaphoreType.DMA,
            pltpu.SemaphoreType.DMA,
        ],
    )
    def k(g_hbm, row_hbm, col_hbm, z_hbm, out_hbm, ridx_v, cidx_v, rows_v,
          rt_v, ct_v, shared, si0, si1, sg0, sg1, ss0, ss1):
        cid = lax.axis_index("c")
        sid = lax.axis_index("s")
        wid = sid * NC + cid
        base = wid * EPW
        sem_i = (si0, si1)
        sem_g = (sg0, sg1)
        sem_s = (ss0, ss1)

        _zero_shared(shared, z_hbm, sid)
        plsc.subcore_barrier()

        def idx_start(slot, c):
            pltpu.async_copy(row_hbm.at[pl.ds(base + c * CH, CH)],
                             ridx_v.at[slot], sem_i[slot])
            pltpu.async_copy(col_hbm.at[pl.ds(base + c * CH, CH)],
                             cidx_v.at[slot], sem_i[slot])

        def idx_wait(slot, c):
            pltpu.make_async_copy(row_hbm.at[pl.ds(base + c * CH, CH)],
                                  ridx_v.at[slot], sem_i[slot]).wait()
            pltpu.make_async_copy(col_hbm.at[pl.ds(base + c * CH, CH)],
                                  cidx_v.at[slot], sem_i[slot]).wait()

        def gather_start(slot):
            pltpu.async_copy(g_hbm.at[ridx_v.at[slot]], rows_v.at[slot],
                             sem_g[slot])

        def gather_wait(slot):
            pltpu.make_async_copy(g_hbm.at[ridx_v.at[slot]], rows_v.at[slot],
                                  sem_g[slot]).wait()

        def scatter_start(slot):
            pltpu.async_copy(rows_v.at[slot], shared.at[cidx_v.at[slot]],
                             sem_s[slot], add=True)

        def scatter_wait(slot):
            pltpu.make_async_copy(rows_v.at[slot], shared.at[cidx_v.at[slot]],
                                  sem_s[slot]).wait()

        idx_start(0, 0)
        idx_wait(0, 0)
        gather_start(0)

        @pl.loop(0, NFULL // NBUF)
        def _(it):
            for p in range(NBUF):
                b = p
                o = 1 - p
                c = it * NBUF + p

                @pl.when(c + 1 < NFULL)
                def _():
                    @pl.when(c >= 1)
                    def _():
                        scatter_wait(o)
                    idx_start(o, c + 1)
                    idx_wait(o, c + 1)
                    gather_start(o)

                gather_wait(b)
                scatter_start(b)

        scatter_wait(0)
        scatter_wait(1)

        if TAIL:
            off = base + NFULL * CH
            pltpu.sync_copy(row_hbm.at[pl.ds(off, TAIL)], rt_v)
            pltpu.sync_copy(col_hbm.at[pl.ds(off, TAIL)], ct_v)
            pltpu.sync_copy(g_hbm.at[rt_v], rows_v.at[0, pl.ds(0, TAIL)])
            pltpu.sync_copy(rows_v.at[0, pl.ds(0, TAIL)], shared.at[ct_v],
                            add=True)

        plsc.subcore_barrier()
        _writeout_shared(shared, out_hbm, cid, sid)

    return k(g, row, col, z128)


def _tc_prep(deg_parts, x):
    def body(degp_ref, x_ref, dinv1d_ref, dinvc_ref, g0_ref):
        deg = jnp.sum(degp_ref[...], axis=0, keepdims=True)
        dinv = jnp.where(deg > 0, lax.rsqrt(deg), 0.0)
        dinv1d_ref[...] = dinv.reshape(NPAD)
        dinvc = dinv.reshape(NPAD, 1)[:N]
        dinvc_ref[...] = dinvc
        g0_ref[...] = x_ref[...] * dinvc

    return pl.pallas_call(
        body,
        out_shape=(jax.ShapeDtypeStruct((NPAD,), jnp.float32),
                   jax.ShapeDtypeStruct((N, 1), jnp.float32),
                   jax.ShapeDtypeStruct((N, D), jnp.float32)),
    )(deg_parts, x)


NB = 1000


def _tc_mlp_stats(h, agg_parts, b_parts, dinvc, wm, bm, we1r, we2r):
    def body(h_ref, aggp_ref, bp_ref, dinv_ref, wm_ref, bm_ref, we1_ref,
             we2_ref, h2_ref, st_ref):
        dinv = dinv_ref[...]
        b4 = jnp.sum(bp_ref[...], axis=0)
        ef1 = b4[:, 3:4] * we1_ref[...]
        ef2 = (b4[:, 0:1] * we2_ref[...][0:1]
               + b4[:, 1:2] * we2_ref[...][1:2]
               + b4[:, 2:3] * we2_ref[...][2:3])
        ef = jnp.concatenate([ef1, ef2], axis=1)
        agg = dinv * (aggp_ref[0] + aggp_ref[1] + ef)
        cat = jnp.concatenate([h_ref[...], agg], axis=1)
        z = lax.dot_general(cat, wm_ref[...], (((1,), (1,)), ((), ())),
                            preferred_element_type=jnp.float32)
        h2 = jnp.maximum(z + bm_ref[...], 0.0)
        h2_ref[...] = h2

        @pl.when(pl.program_id(0) == 0)
        def _():
            st_ref[...] = jnp.zeros_like(st_ref)

        st_ref[...] += jnp.concatenate(
            [jnp.sum(h2, axis=0, keepdims=True),
             jnp.sum(h2 * h2, axis=0, keepdims=True)], axis=0)

    return pl.pallas_call(
        body,
        grid=(N // NB,),
        in_specs=[
            pl.BlockSpec((NB, D), lambda i: (i, 0)),
            pl.BlockSpec((NC, NB, D), lambda i: (0, i, 0)),
            pl.BlockSpec((NW, NB, 4), lambda i: (0, i, 0)),
            pl.BlockSpec((NB, 1), lambda i: (i, 0)),
            pl.BlockSpec((D, 2 * D), lambda i: (0, 0)),
            pl.BlockSpec((1, D), lambda i: (0, 0)),
            pl.BlockSpec((1, D // 2), lambda i: (0, 0)),
            pl.BlockSpec((3, D // 2), lambda i: (0, 0)),
        ],
        out_specs=(pl.BlockSpec((NB, D), lambda i: (i, 0)),
                   pl.BlockSpec((2, D), lambda i: (0, 0))),
        out_shape=(jax.ShapeDtypeStruct((N, D), jnp.float32),
                   jax.ShapeDtypeStruct((2, D), jnp.float32)),
    )(h, agg_parts, b_parts, dinvc, wm, bm, we1r, we2r)


def _tc_bn_apply(h2, stats, dinvc, gm, bt, last):
    def body(h2_ref, st_ref, dinv_ref, gm_ref, bt_ref, hout_ref, gout_ref):
        st = st_ref[...]
        mean = st[0:1] / N
        var = st[1:2] / N - mean * mean
        hn = (h2_ref[...] - mean) / jnp.sqrt(var + EPS) * gm_ref[...] \
            + bt_ref[...]
        if not last:
            hn = jnp.maximum(hn, 0.0)
        hout_ref[...] = hn
        gout_ref[...] = hn * dinv_ref[...]

    return pl.pallas_call(
        body,
        grid=(N // NB,),
        in_specs=[
            pl.BlockSpec((NB, D), lambda i: (i, 0)),
            pl.BlockSpec((2, D), lambda i: (0, 0)),
            pl.BlockSpec((NB, 1), lambda i: (i, 0)),
            pl.BlockSpec((1, D), lambda i: (0, 0)),
            pl.BlockSpec((1, D), lambda i: (0, 0)),
        ],
        out_specs=(pl.BlockSpec((NB, D), lambda i: (i, 0)),
                   pl.BlockSpec((NB, D), lambda i: (i, 0))),
        out_shape=(jax.ShapeDtypeStruct((N, D), jnp.float32),
                   jax.ShapeDtypeStruct((N, D), jnp.float32)),
    )(h2, stats, dinvc, gm, bt)


def kernel(x, edge_attr, Wmlp, bmlp, We1, We2, gamma, beta, edge_index):
    row = edge_index[0]
    col = edge_index[1]
    ea_bf = edge_attr.astype(jnp.bfloat16).astype(jnp.float32)
    ea_pad = jnp.concatenate(
        [ea_bf, jnp.zeros((E, 12), jnp.float32)], axis=1)
    zn = jnp.zeros((NPAD,), jnp.float32)
    z4 = jnp.zeros((4 * NPAD,), jnp.float32)
    z128 = jnp.zeros((RPT, D), jnp.float32)

    deg_parts = _sc_deg(col, zn)
    dinv1d, dinvc, g = _tc_prep(deg_parts, x)
    b_parts = _sc_bacc(ea_pad, row, col, dinv1d, z4)
    b_parts = b_parts.reshape(NW, NPAD, 4)

    h = x
    for l in range(L):
        agg_parts = _sc_spmm(g, row, col, z128)
        h2, stats = _tc_mlp_stats(
            h, agg_parts, b_parts, dinvc,
            Wmlp[l], bmlp[l].reshape(1, D),
            We1[l].reshape(1, D // 2).astype(jnp.bfloat16).astype(jnp.float32),
            We2[l].T.astype(jnp.bfloat16).astype(jnp.float32))
        h, g = _tc_bn_apply(h2, stats, dinvc,
                            gamma[l].reshape(1, D), beta[l].reshape(1, D),
                            last=(l == L - 1))
    return h
